# Initial kernel scaffold; baseline (speedup 1.0000x reference)
import jax
import jax.numpy as jnp
from jax.experimental import pallas as pl


def kernel(input, W1, b1, W2, b2, ws, target_joint_mask, target_heading):
    raise NotImplementedError("write your pallas kernel here")


# speedup vs baseline: 1.7634x; 1.7634x over previous
import jax
import jax.numpy as jnp
from jax.experimental import pallas as pl


def _body(J, x_ref, m_ref, ws_ref, W1_ref, b1_ref, W2_ref, b2_ref, out_ref):
    mw = m_ref[...] * ws_ref[...]
    acc = jnp.dot(mw, b2_ref[...], preferred_element_type=jnp.float32)
    for j in range(J):
        xj = x_ref[j]
        h = jnp.dot(xj, W1_ref[j], preferred_element_type=jnp.float32)
        h = h + b1_ref[j : j + 1, :]
        s = h * (1.0 / (1.0 + jnp.exp(-h)))
        a = (s * mw[:, j : j + 1]).astype(jnp.bfloat16)
        acc = acc + jnp.dot(a, W2_ref[j], preferred_element_type=jnp.float32)
    out_ref[...] = acc


def kernel(input, W1, b1, W2, b2, ws, target_joint_mask, target_heading):
    B, J, _ = input.shape
    D = b1.shape[1]
    BB = 512
    mask_f = jnp.concatenate(
        [target_joint_mask, target_heading[:, None]], axis=1
    ).astype(jnp.float32)
    ws2d = ws.reshape(1, J)
    xt = jnp.transpose(input, (1, 0, 2))
    W2b = W2.astype(jnp.bfloat16)

    import functools

    body = functools.partial(_body, J)
    out = pl.pallas_call(
        body,
        grid=(B // BB,),
        in_specs=[
            pl.BlockSpec((J, BB, 3), lambda i: (0, i, 0)),
            pl.BlockSpec((BB, J), lambda i: (i, 0)),
            pl.BlockSpec((1, J), lambda i: (0, 0)),
            pl.BlockSpec((J, 3, D), lambda i: (0, 0, 0)),
            pl.BlockSpec((J, D), lambda i: (0, 0)),
            pl.BlockSpec((J, D, D), lambda i: (0, 0, 0)),
            pl.BlockSpec((J, D), lambda i: (0, 0)),
        ],
        out_specs=pl.BlockSpec((BB, D), lambda i: (i, 0)),
        out_shape=jax.ShapeDtypeStruct((B, D), jnp.float32),
    )(xt, mask_f, ws2d, W1, b1, W2b, b2)
    return out
